# baseline (speedup 1.0000x reference)
import jax
import jax.numpy as jnp
from jax.experimental import pallas as pl
from jax.experimental.pallas import tpu as pltpu

B, S, H = 128, 577, 1024
E = 8
NEG_BIG = -3.0e38


def _router_kernel(vf_hbm, w_ref, b_ref, rw_ref, se_ref, cls_vmem, sem):
    cps = [
        pltpu.make_async_copy(
            vf_hbm.at[pl.ds(i, 1), 0:H], cls_vmem.at[pl.ds(i, 1), :], sem)
        for i in range(B)
    ]
    for cp in cps:
        cp.start()
    for cp in cps:
        cp.wait()
    cls = cls_vmem[...]
    w = w_ref[...]
    logits = jax.lax.dot_general(
        cls, w, (((1,), (1,)), ((), ())),
        preferred_element_type=jnp.float32) + b_ref[...]
    idx = jax.lax.broadcasted_iota(jnp.int32, (B, E), 1)
    m1 = jnp.max(logits, axis=1, keepdims=True)
    i1 = jnp.min(jnp.where(logits == m1, idx, E), axis=1, keepdims=True)
    masked = jnp.where(idx == i1, NEG_BIG, logits)
    m2 = jnp.max(masked, axis=1, keepdims=True)
    i2 = jnp.min(jnp.where(masked == m2, idx, E), axis=1, keepdims=True)
    e = jnp.exp(m2 - m1)
    w1 = 1.0 / (1.0 + e)
    rw_ref[...] = jnp.concatenate([w1, 1.0 - w1], axis=1)
    se_ref[...] = jnp.concatenate([i1, i2], axis=1)


def kernel(vision_features, W, b):
    return pl.pallas_call(
        _router_kernel,
        out_shape=(
            jax.ShapeDtypeStruct((B, 2), jnp.float32),
            jax.ShapeDtypeStruct((B, 2), jnp.int32),
        ),
        grid=(1,),
        in_specs=[
            pl.BlockSpec(memory_space=pl.ANY),
            pl.BlockSpec((E, H), lambda i: (0, 0)),
            pl.BlockSpec((1, E), lambda i: (0, 0)),
        ],
        out_specs=(
            pl.BlockSpec((B, 2), lambda i: (0, 0)),
            pl.BlockSpec((B, 2), lambda i: (0, 0)),
        ),
        scratch_shapes=[
            pltpu.VMEM((B, H), jnp.float32),
            pltpu.SemaphoreType.DMA,
        ],
    )(vision_features.reshape(B, S * H), W, b.reshape(1, E))


# speedup vs baseline: 1.0152x; 1.0152x over previous
import jax
import jax.numpy as jnp
from jax import lax
from jax.experimental import pallas as pl
from jax.experimental.pallas import tpu as pltpu
from jax.experimental.pallas import tpu_sc as plsc

B, S, H = 128, 577, 1024
E = 8
L = 16
NC, NS = 2, 16
NW = NC * NS
RPW = B // NW
NCHUNK = H // L
NEG_BIG = -3.0e38

_GATHER_DN = lax.GatherDimensionNumbers(
    offset_dims=(), collapsed_slice_dims=(0,), start_index_map=(0,))


def _bcast(x, i):
    return lax.gather(x, jnp.full((L, 1), i, jnp.int32), _GATHER_DN, (1,),
                      mode=lax.GatherScatterMode.PROMISE_IN_BOUNDS)


def _round_bf16(x):
    u = plsc.bitcast(x, jnp.int32)
    odd = jnp.bitwise_and(lax.shift_right_logical(u, 16), 1)
    u = jnp.bitwise_and(u + 0x7FFF + odd, jnp.int32(-65536))
    return plsc.bitcast(u, jnp.float32)


def _router_body(vf_hbm, w_hbm, b_hbm, rw_hbm, se_hbm,
                 rows, wv, bvec, oww, oii, sem):
    wid = lax.axis_index("s") * NC + lax.axis_index("c")

    cps = [pltpu.make_async_copy(w_hbm.at[:, :], wv.at[:, :], sem),
           pltpu.make_async_copy(b_hbm.at[pl.ds(0, E)], bvec.at[pl.ds(0, E)],
                                 sem)]
    for j in range(RPW):
        ridx = (wid * RPW + j) * S
        cps.append(pltpu.make_async_copy(vf_hbm.at[ridx], rows.at[j], sem))
    for cp in cps:
        cp.start()
    for cp in cps:
        cp.wait()

    acc = [[jnp.zeros((L,), jnp.float32) for _ in range(E)]
           for _ in range(RPW)]
    for c in range(NCHUNK):
        wchunks = [wv[e, pl.ds(c * L, L)] for e in range(E)]
        for r in range(RPW):
            rchunk = _round_bf16(rows[r, pl.ds(c * L, L)])
            for e in range(E):
                acc[r][e] = acc[r][e] + rchunk * wchunks[e]

    lane = lax.iota(jnp.int32, L)
    bv = bvec[...]
    out_w = jnp.zeros((L,), jnp.float32)
    out_i = jnp.zeros((L,), jnp.int32)
    for r in range(RPW):
        lv = jnp.zeros((L,), jnp.float32)
        for e in range(E):
            tot = _bcast(plsc.cumsum(acc[r][e]), L - 1)
            lv = jnp.where(lane == e, tot, lv)
        keys = jnp.where(lane < E, lv + bv, NEG_BIG)
        sk, sv = plsc.sort_key_val(keys, lane, descending=True)
        m1 = _bcast(sk, 0)
        m2 = _bcast(sk, 1)
        i1 = _bcast(sv, 0)
        i2 = _bcast(sv, 1)
        ev = jnp.exp(m2 - m1)
        w1 = 1.0 / (1.0 + ev)
        out_w = jnp.where(lane == 2 * r, w1, out_w)
        out_w = jnp.where(lane == 2 * r + 1, 1.0 - w1, out_w)
        out_i = jnp.where(lane == 2 * r, i1, out_i)
        out_i = jnp.where(lane == 2 * r + 1, i2, out_i)

    oww[...] = out_w
    oii[...] = out_i
    pltpu.sync_copy(oww.at[pl.ds(0, 2 * RPW)],
                    rw_hbm.at[pl.ds(wid * 2 * RPW, 2 * RPW)])
    pltpu.sync_copy(oii.at[pl.ds(0, 2 * RPW)],
                    se_hbm.at[pl.ds(wid * 2 * RPW, 2 * RPW)])


def kernel(vision_features, W, b):
    mesh = plsc.VectorSubcoreMesh(core_axis_name="c", subcore_axis_name="s",
                                  num_cores=NC)
    rw_flat, se_flat = pl.kernel(
        _router_body,
        out_type=(
            jax.ShapeDtypeStruct((B * 2,), jnp.float32),
            jax.ShapeDtypeStruct((B * 2,), jnp.int32),
        ),
        mesh=mesh,
        compiler_params=pltpu.CompilerParams(needs_layout_passes=False),
        scratch_types=[
            pltpu.VMEM((RPW, H), jnp.float32),
            pltpu.VMEM((E, H), jnp.float32),
            pltpu.VMEM((L,), jnp.float32),
            pltpu.VMEM((L,), jnp.float32),
            pltpu.VMEM((L,), jnp.int32),
            pltpu.SemaphoreType.DMA,
        ],
    )(vision_features.reshape(B * S, H),
      W.astype(jnp.bfloat16).astype(jnp.float32), b)
    return rw_flat.reshape(B, 2), se_flat.reshape(B, 2)


# speedup vs baseline: 1.0188x; 1.0035x over previous
import jax
import jax.numpy as jnp
from jax import lax
from jax.experimental import pallas as pl
from jax.experimental.pallas import tpu as pltpu
from jax.experimental.pallas import tpu_sc as plsc

B, S, H = 128, 577, 1024
E = 8
L = 16
NC, NS = 2, 16
NW = NC * NS
RPW = B // NW
NCHUNK = H // L
NEG_BIG = -3.0e38

_GATHER_DN = lax.GatherDimensionNumbers(
    offset_dims=(), collapsed_slice_dims=(0,), start_index_map=(0,))


def _bcast(x, i):
    return lax.gather(x, jnp.full((L, 1), i, jnp.int32), _GATHER_DN, (1,),
                      mode=lax.GatherScatterMode.PROMISE_IN_BOUNDS)


def _round_bf16(x):
    u = plsc.bitcast(x, jnp.int32)
    odd = jnp.bitwise_and(lax.shift_right_logical(u, 16), 1)
    u = jnp.bitwise_and(u + 0x7FFF + odd, jnp.int32(-65536))
    return plsc.bitcast(u, jnp.float32)


def _router_body(vf_hbm, w_hbm, b_hbm, rw_hbm, se_hbm,
                 rows, wv, bvec, oww, oii, sem):
    wid = lax.axis_index("s") * NC + lax.axis_index("c")

    cps = [pltpu.make_async_copy(w_hbm.at[:, :], wv.at[:, :], sem),
           pltpu.make_async_copy(b_hbm.at[pl.ds(0, E)], bvec.at[pl.ds(0, E)],
                                 sem)]
    for j in range(RPW):
        ridx = (wid * RPW + j) * S
        cps.append(pltpu.make_async_copy(vf_hbm.at[ridx], rows.at[j], sem))
    for cp in cps:
        cp.start()
    for cp in cps:
        cp.wait()

    acc = [[jnp.zeros((L,), jnp.float32) for _ in range(E)]
           for _ in range(RPW)]
    for c in range(NCHUNK):
        wchunks = [wv[e, pl.ds(c * L, L)] for e in range(E)]
        for r in range(RPW):
            rchunk = _round_bf16(rows[r, pl.ds(c * L, L)])
            for e in range(E):
                acc[r][e] = acc[r][e] + rchunk * wchunks[e]

    lane = lax.iota(jnp.int32, L)
    bv = bvec[...]
    out_w = jnp.zeros((L,), jnp.float32)
    out_i = jnp.zeros((L,), jnp.int32)
    for r in range(RPW):
        lv = jnp.zeros((L,), jnp.float32)
        for e in range(E):
            tot = _bcast(plsc.cumsum(acc[r][e]), L - 1)
            lv = jnp.where(lane == e, tot, lv)
        keys = jnp.where(lane < E, lv + bv, NEG_BIG)
        sk, sv = plsc.sort_key_val(keys, lane, descending=True)
        m1 = _bcast(sk, 0)
        m2 = _bcast(sk, 1)
        i1 = _bcast(sv, 0)
        i2 = _bcast(sv, 1)
        ev = jnp.exp(m2 - m1)
        w1 = 1.0 / (1.0 + ev)
        out_w = jnp.where(lane == 2 * r, w1, out_w)
        out_w = jnp.where(lane == 2 * r + 1, 1.0 - w1, out_w)
        out_i = jnp.where(lane == 2 * r, i1, out_i)
        out_i = jnp.where(lane == 2 * r + 1, i2, out_i)

    oww[...] = out_w
    oii[...] = out_i
    pltpu.sync_copy(oww.at[pl.ds(0, 2 * RPW)],
                    rw_hbm.at[pl.ds(wid * 2 * RPW, 2 * RPW)])
    pltpu.sync_copy(oii.at[pl.ds(0, 2 * RPW)],
                    se_hbm.at[pl.ds(wid * 2 * RPW, 2 * RPW)])


def kernel(vision_features, W, b):
    mesh = plsc.VectorSubcoreMesh(core_axis_name="c", subcore_axis_name="s",
                                  num_cores=NC)
    rw_flat, se_flat = pl.kernel(
        _router_body,
        out_type=(
            jax.ShapeDtypeStruct((B * 2,), jnp.float32),
            jax.ShapeDtypeStruct((B * 2,), jnp.int32),
        ),
        mesh=mesh,
        compiler_params=pltpu.CompilerParams(needs_layout_passes=False,
                                             use_tc_tiling_on_sc=True),
        scratch_types=[
            pltpu.VMEM((RPW, H), jnp.float32),
            pltpu.VMEM((E, H), jnp.float32),
            pltpu.VMEM((L,), jnp.float32),
            pltpu.VMEM((L,), jnp.float32),
            pltpu.VMEM((L,), jnp.int32),
            pltpu.SemaphoreType.DMA,
        ],
    )(vision_features.reshape(B * S, H),
      W.astype(jnp.bfloat16).astype(jnp.float32), b)
    return rw_flat.reshape(B, 2), se_flat.reshape(B, 2)


# speedup vs baseline: 1.8341x; 1.8002x over previous
import jax
import jax.numpy as jnp
from jax import lax
from jax.experimental import pallas as pl
from jax.experimental.pallas import tpu as pltpu
from jax.experimental.pallas import tpu_sc as plsc

B, S, H = 128, 577, 1024
E = 8
L = 16
NC, NS = 2, 16
NW = NC * NS
RPW = B // NW
NCHUNK = H // L
NEG_BIG = -3.0e38

_GATHER_DN = lax.GatherDimensionNumbers(
    offset_dims=(), collapsed_slice_dims=(0,), start_index_map=(0,))


def _bcast(x, i):
    return lax.gather(x, jnp.full((L, 1), i, jnp.int32), _GATHER_DN, (1,),
                      mode=lax.GatherScatterMode.PROMISE_IN_BOUNDS)


def _round_bf16(x):
    u = plsc.bitcast(x, jnp.int32)
    odd = jnp.bitwise_and(lax.shift_right_logical(u, 16), 1)
    u = jnp.bitwise_and(u + 0x7FFF + odd, jnp.int32(-65536))
    return plsc.bitcast(u, jnp.float32)


def _router_body(vf_hbm, w_hbm, b_hbm, rw_hbm, se_hbm,
                 rows, wv, bvec, oww, oii, sem):
    wid = lax.axis_index("s") * NC + lax.axis_index("c")

    cps = [pltpu.make_async_copy(w_hbm.at[:, :], wv.at[:, :], sem),
           pltpu.make_async_copy(b_hbm.at[pl.ds(0, E)], bvec.at[pl.ds(0, E)],
                                 sem)]
    for j in range(RPW):
        bidx = wid * RPW + j
        cps.append(pltpu.make_async_copy(vf_hbm.at[bidx, 0], rows.at[j], sem))
    for cp in cps:
        cp.start()
    for cp in cps:
        cp.wait()

    acc = [[jnp.zeros((L,), jnp.float32) for _ in range(E)]
           for _ in range(RPW)]
    for c in range(NCHUNK):
        wchunks = [wv[e, pl.ds(c * L, L)] for e in range(E)]
        for r in range(RPW):
            rchunk = _round_bf16(rows[r, pl.ds(c * L, L)])
            for e in range(E):
                acc[r][e] = acc[r][e] + rchunk * wchunks[e]

    lane = lax.iota(jnp.int32, L)
    bv = bvec[...]
    out_w = jnp.zeros((L,), jnp.float32)
    out_i = jnp.zeros((L,), jnp.int32)
    for r in range(RPW):
        lv = jnp.zeros((L,), jnp.float32)
        for e in range(E):
            tot = _bcast(plsc.cumsum(acc[r][e]), L - 1)
            lv = jnp.where(lane == e, tot, lv)
        keys = jnp.where(lane < E, lv + bv, NEG_BIG)
        sk, sv = plsc.sort_key_val(keys, lane, descending=True)
        m1 = _bcast(sk, 0)
        m2 = _bcast(sk, 1)
        i1 = _bcast(sv, 0)
        i2 = _bcast(sv, 1)
        ev = jnp.exp(m2 - m1)
        w1 = 1.0 / (1.0 + ev)
        out_w = jnp.where(lane == 2 * r, w1, out_w)
        out_w = jnp.where(lane == 2 * r + 1, 1.0 - w1, out_w)
        out_i = jnp.where(lane == 2 * r, i1, out_i)
        out_i = jnp.where(lane == 2 * r + 1, i2, out_i)

    oww[...] = out_w
    oii[...] = out_i
    pltpu.sync_copy(oww.at[pl.ds(0, 2 * RPW)],
                    rw_hbm.at[pl.ds(wid * 2 * RPW, 2 * RPW)])
    pltpu.sync_copy(oii.at[pl.ds(0, 2 * RPW)],
                    se_hbm.at[pl.ds(wid * 2 * RPW, 2 * RPW)])


def kernel(vision_features, W, b):
    mesh = plsc.VectorSubcoreMesh(core_axis_name="c", subcore_axis_name="s",
                                  num_cores=NC)
    rw_flat, se_flat = pl.kernel(
        _router_body,
        out_type=(
            jax.ShapeDtypeStruct((B * 2,), jnp.float32),
            jax.ShapeDtypeStruct((B * 2,), jnp.int32),
        ),
        mesh=mesh,
        compiler_params=pltpu.CompilerParams(needs_layout_passes=False,
                                             use_tc_tiling_on_sc=True),
        scratch_types=[
            pltpu.VMEM((RPW, H), jnp.float32),
            pltpu.VMEM((E, H), jnp.float32),
            pltpu.VMEM((L,), jnp.float32),
            pltpu.VMEM((L,), jnp.float32),
            pltpu.VMEM((L,), jnp.int32),
            pltpu.SemaphoreType.DMA,
        ],
    )(vision_features, W.astype(jnp.bfloat16).astype(jnp.float32), b)
    return rw_flat.reshape(B, 2), se_flat.reshape(B, 2)


# speedup vs baseline: 1.8440x; 1.0054x over previous
import jax
import jax.numpy as jnp
from jax import lax
from jax.experimental import pallas as pl
from jax.experimental.pallas import tpu as pltpu
from jax.experimental.pallas import tpu_sc as plsc

B, S, H = 128, 577, 1024
E = 8
L = 16
NC, NS = 2, 16
NW = NC * NS
RPW = B // NW
NCHUNK = H // L
NEG_BIG = -3.0e38

_GATHER_DN = lax.GatherDimensionNumbers(
    offset_dims=(), collapsed_slice_dims=(0,), start_index_map=(0,))


def _bcast(x, i):
    return lax.gather(x, jnp.full((L, 1), i, jnp.int32), _GATHER_DN, (1,),
                      mode=lax.GatherScatterMode.PROMISE_IN_BOUNDS)


def _round_bf16(x):
    u = plsc.bitcast(x, jnp.int32)
    odd = jnp.bitwise_and(lax.shift_right_logical(u, 16), 1)
    u = jnp.bitwise_and(u + 0x7FFF + odd, jnp.int32(-65536))
    return plsc.bitcast(u, jnp.float32)


def _router_body(vf_hbm, w_hbm, b_hbm, rw_hbm, se_hbm,
                 rows, wv, bvec, oww, oii, sem):
    wid = lax.axis_index("s") * NC + lax.axis_index("c")

    cps = [pltpu.make_async_copy(w_hbm.at[:, :], wv.at[:, :], sem),
           pltpu.make_async_copy(b_hbm.at[pl.ds(0, E)], bvec.at[pl.ds(0, E)],
                                 sem)]
    for j in range(RPW):
        bidx = wid * RPW + j
        cps.append(pltpu.make_async_copy(vf_hbm.at[bidx, 0], rows.at[j], sem))
    for cp in cps:
        cp.start()
    for cp in cps:
        cp.wait()

    def chunk_body(c, accs):
        accs = list(accs)
        base = c * L
        wch = [wv[e, pl.ds(base, L)] for e in range(E)]
        k = 0
        for r in range(RPW):
            rch = _round_bf16(rows[r, pl.ds(base, L)])
            for e in range(E):
                accs[k] = accs[k] + rch * wch[e]
                k += 1
        return tuple(accs)

    accs = lax.fori_loop(
        0, NCHUNK, chunk_body,
        tuple(jnp.zeros((L,), jnp.float32) for _ in range(RPW * E)))
    acc = [[accs[r * E + e] for e in range(E)] for r in range(RPW)]

    lane = lax.iota(jnp.int32, L)
    bv = bvec[...]
    out_w = jnp.zeros((L,), jnp.float32)
    out_i = jnp.zeros((L,), jnp.int32)
    for r in range(RPW):
        lv = jnp.zeros((L,), jnp.float32)
        for e in range(E):
            tot = _bcast(plsc.cumsum(acc[r][e]), L - 1)
            lv = jnp.where(lane == e, tot, lv)
        keys = jnp.where(lane < E, lv + bv, NEG_BIG)
        sk, sv = plsc.sort_key_val(keys, lane, descending=True)
        m1 = _bcast(sk, 0)
        m2 = _bcast(sk, 1)
        i1 = _bcast(sv, 0)
        i2 = _bcast(sv, 1)
        ev = jnp.exp(m2 - m1)
        w1 = 1.0 / (1.0 + ev)
        out_w = jnp.where(lane == 2 * r, w1, out_w)
        out_w = jnp.where(lane == 2 * r + 1, 1.0 - w1, out_w)
        out_i = jnp.where(lane == 2 * r, i1, out_i)
        out_i = jnp.where(lane == 2 * r + 1, i2, out_i)

    oww[...] = out_w
    oii[...] = out_i
    pltpu.sync_copy(oww.at[pl.ds(0, 2 * RPW)],
                    rw_hbm.at[pl.ds(wid * 2 * RPW, 2 * RPW)])
    pltpu.sync_copy(oii.at[pl.ds(0, 2 * RPW)],
                    se_hbm.at[pl.ds(wid * 2 * RPW, 2 * RPW)])


def kernel(vision_features, W, b):
    mesh = plsc.VectorSubcoreMesh(core_axis_name="c", subcore_axis_name="s",
                                  num_cores=NC)
    rw_flat, se_flat = pl.kernel(
        _router_body,
        out_type=(
            jax.ShapeDtypeStruct((B * 2,), jnp.float32),
            jax.ShapeDtypeStruct((B * 2,), jnp.int32),
        ),
        mesh=mesh,
        compiler_params=pltpu.CompilerParams(needs_layout_passes=False,
                                             use_tc_tiling_on_sc=True),
        scratch_types=[
            pltpu.VMEM((RPW, H), jnp.float32),
            pltpu.VMEM((E, H), jnp.float32),
            pltpu.VMEM((L,), jnp.float32),
            pltpu.VMEM((L,), jnp.float32),
            pltpu.VMEM((L,), jnp.int32),
            pltpu.SemaphoreType.DMA,
        ],
    )(vision_features, W.astype(jnp.bfloat16).astype(jnp.float32), b)
    return rw_flat.reshape(B, 2), se_flat.reshape(B, 2)


# speedup vs baseline: 15.4042x; 8.3537x over previous
import jax
import jax.numpy as jnp
from jax import lax
from jax.experimental import pallas as pl
from jax.experimental.pallas import tpu as pltpu
from jax.experimental.pallas import tpu_sc as plsc

B, S, H = 128, 577, 1024
E = 8
L = 16
NC, NS = 2, 16
NW = NC * NS
RPW = B // NW
NCHUNK = H // L
NEG_BIG = -3.0e38

_GATHER_DN = lax.GatherDimensionNumbers(
    offset_dims=(), collapsed_slice_dims=(0,), start_index_map=(0,))


def _bcast(x, i):
    return lax.gather(x, jnp.full((L, 1), i, jnp.int32), _GATHER_DN, (1,),
                      mode=lax.GatherScatterMode.PROMISE_IN_BOUNDS)


def _round_bf16(x):
    u = plsc.bitcast(x, jnp.int32)
    odd = jnp.bitwise_and(lax.shift_right_logical(u, 16), 1)
    u = jnp.bitwise_and(u + 0x7FFF + odd, jnp.int32(-65536))
    return plsc.bitcast(u, jnp.float32)


def _router_body(vf_hbm, w_hbm, b_hbm, rw_hbm, se_hbm,
                 rows, wv, bvec, oww, oii, sem):
    wid = lax.axis_index("s") * NC + lax.axis_index("c")

    cps = [pltpu.make_async_copy(w_hbm.at[:, :], wv.at[:, :], sem),
           pltpu.make_async_copy(b_hbm.at[pl.ds(0, E)], bvec.at[pl.ds(0, E)],
                                 sem)]
    for j in range(RPW):
        bidx = wid * RPW + j
        cps.append(pltpu.make_async_copy(vf_hbm.at[0, bidx], rows.at[j], sem))
    for cp in cps:
        cp.start()
    for cp in cps:
        cp.wait()

    acc = [[jnp.zeros((L,), jnp.float32) for _ in range(E)]
           for _ in range(RPW)]
    for c in range(NCHUNK):
        wchunks = [wv[e, pl.ds(c * L, L)] for e in range(E)]
        for r in range(RPW):
            rchunk = _round_bf16(rows[r, pl.ds(c * L, L)])
            for e in range(E):
                acc[r][e] = acc[r][e] + rchunk * wchunks[e]

    lane = lax.iota(jnp.int32, L)
    bv = bvec[...]
    out_w = jnp.zeros((L,), jnp.float32)
    out_i = jnp.zeros((L,), jnp.int32)
    for r in range(RPW):
        lv = jnp.zeros((L,), jnp.float32)
        for e in range(E):
            tot = _bcast(plsc.cumsum(acc[r][e]), L - 1)
            lv = jnp.where(lane == e, tot, lv)
        keys = jnp.where(lane < E, lv + bv, NEG_BIG)
        sk, sv = plsc.sort_key_val(keys, lane, descending=True)
        m1 = _bcast(sk, 0)
        m2 = _bcast(sk, 1)
        i1 = _bcast(sv, 0)
        i2 = _bcast(sv, 1)
        ev = jnp.exp(m2 - m1)
        w1 = 1.0 / (1.0 + ev)
        out_w = jnp.where(lane == 2 * r, w1, out_w)
        out_w = jnp.where(lane == 2 * r + 1, 1.0 - w1, out_w)
        out_i = jnp.where(lane == 2 * r, i1, out_i)
        out_i = jnp.where(lane == 2 * r + 1, i2, out_i)

    oww[...] = out_w
    oii[...] = out_i
    pltpu.sync_copy(oww.at[pl.ds(0, 2 * RPW)],
                    rw_hbm.at[pl.ds(wid * 2 * RPW, 2 * RPW)])
    pltpu.sync_copy(oii.at[pl.ds(0, 2 * RPW)],
                    se_hbm.at[pl.ds(wid * 2 * RPW, 2 * RPW)])


def kernel(vision_features, W, b):
    mesh = plsc.VectorSubcoreMesh(core_axis_name="c", subcore_axis_name="s",
                                  num_cores=NC)
    rw_flat, se_flat = pl.kernel(
        _router_body,
        out_type=(
            jax.ShapeDtypeStruct((B * 2,), jnp.float32),
            jax.ShapeDtypeStruct((B * 2,), jnp.int32),
        ),
        mesh=mesh,
        compiler_params=pltpu.CompilerParams(needs_layout_passes=False,
                                             use_tc_tiling_on_sc=True),
        scratch_types=[
            pltpu.VMEM((RPW, H), jnp.float32),
            pltpu.VMEM((E, H), jnp.float32),
            pltpu.VMEM((L,), jnp.float32),
            pltpu.VMEM((L,), jnp.float32),
            pltpu.VMEM((L,), jnp.int32),
            pltpu.SemaphoreType.DMA,
        ],
    )(vision_features.transpose(1, 0, 2),
      W.astype(jnp.bfloat16).astype(jnp.float32), b)
    return rw_flat.reshape(B, 2), se_flat.reshape(B, 2)


# speedup vs baseline: 81.7507x; 5.3070x over previous
import jax
import jax.numpy as jnp
from jax.experimental import pallas as pl

B, S, H = 128, 577, 1024
E = 8
NEG_BIG = -3.0e38


def _router_kernel(vf_ref, w_ref, b_ref, rw_ref, se_ref):
    cls = vf_ref[0]
    w = w_ref[...]
    logits = jax.lax.dot_general(
        cls, w, (((1,), (1,)), ((), ())),
        preferred_element_type=jnp.float32) + b_ref[...]
    idx = jax.lax.broadcasted_iota(jnp.int32, (B, E), 1)
    m1 = jnp.max(logits, axis=1, keepdims=True)
    i1 = jnp.min(jnp.where(logits == m1, idx, E), axis=1, keepdims=True)
    masked = jnp.where(idx == i1, NEG_BIG, logits)
    m2 = jnp.max(masked, axis=1, keepdims=True)
    i2 = jnp.min(jnp.where(masked == m2, idx, E), axis=1, keepdims=True)
    e = jnp.exp(m2 - m1)
    w1 = 1.0 / (1.0 + e)
    rw_ref[...] = jnp.concatenate([w1, 1.0 - w1], axis=1)
    se_ref[...] = jnp.concatenate([i1, i2], axis=1)


def kernel(vision_features, W, b):
    return pl.pallas_call(
        _router_kernel,
        out_shape=(
            jax.ShapeDtypeStruct((B, 2), jnp.float32),
            jax.ShapeDtypeStruct((B, 2), jnp.int32),
        ),
        grid=(1,),
        in_specs=[
            pl.BlockSpec((1, B, H), lambda i: (0, 0, 0)),
            pl.BlockSpec((E, H), lambda i: (0, 0)),
            pl.BlockSpec((1, E), lambda i: (0, 0)),
        ],
        out_specs=(
            pl.BlockSpec((B, 2), lambda i: (0, 0)),
            pl.BlockSpec((B, 2), lambda i: (0, 0)),
        ),
    )(vision_features.transpose(1, 0, 2), W, b.reshape(1, E))
